# baseline (speedup 1.0000x reference)
import functools

import jax
import jax.numpy as jnp
from jax import lax
from jax.experimental import pallas as pl
from jax.experimental.pallas import tpu as pltpu
from jax.experimental.pallas import tpu_sc as plsc

PAD = 0
VOCAB = 100000
BATCH = 1024
NC, NS, L = 2, 16, 16
NW = NC * NS
ROWS_PER_W = BATCH // NW
FLAT = BATCH * VOCAB
WORK_PER_W = FLAT // NW
ZWORDS = 128000
NFILL = WORK_PER_W // ZWORDS

_mesh = plsc.VectorSubcoreMesh(core_axis_name="c", subcore_axis_name="s")


@functools.partial(
    pl.kernel,
    out_type=jax.ShapeDtypeStruct((FLAT,), jnp.float32),
    mesh=_mesh,
    scratch_types=[
        pltpu.VMEM((ZWORDS,), jnp.float32),
        pltpu.VMEM((ROWS_PER_W,), jnp.int32),
        pltpu.VMEM((ROWS_PER_W,), jnp.int32),
        pltpu.VMEM((ROWS_PER_W,), jnp.float32),
        pltpu.SemaphoreType.DMA,
    ],
)
def _sc_onehot(ids_hbm, out_hbm, zbuf, ids_v, idx_v, val_v, sem):
    wid = lax.axis_index("c") * NS + lax.axis_index("s")
    base_row = wid * ROWS_PER_W
    flat_base = wid * WORK_PER_W

    zero16 = jnp.zeros((L,), jnp.float32)

    def zloop(i, carry):
        for u in range(8):
            zbuf[pl.ds((i * 8 + u) * L, L)] = zero16
        return carry

    lax.fori_loop(0, ZWORDS // (8 * L), zloop, 0)

    pltpu.sync_copy(ids_hbm.at[pl.ds(base_row, ROWS_PER_W)], ids_v)

    def fire(i, carry):
        pltpu.async_copy(
            zbuf, out_hbm.at[pl.ds(flat_base + i * ZWORDS, ZWORDS)], sem
        )
        return carry

    lax.fori_loop(0, NFILL, fire, 0)

    iota = lax.iota(jnp.int32, L)
    for c in range(ROWS_PER_W // L):
        t = ids_v[pl.ds(c * L, L)]
        rows = base_row + c * L + iota
        idx_v[pl.ds(c * L, L)] = rows * VOCAB + t
        val_v[pl.ds(c * L, L)] = jnp.where(t != PAD, 1.0, 0.0).astype(
            jnp.float32
        )

    def drain(i, carry):
        pltpu.make_async_copy(
            zbuf, out_hbm.at[pl.ds(flat_base + i * ZWORDS, ZWORDS)], sem
        ).wait()
        return carry

    lax.fori_loop(0, NFILL, drain, 0)

    pltpu.async_copy(val_v, out_hbm.at[idx_v], sem).wait()


@jax.jit
def kernel(trg_token_ids_batch):
    ids = trg_token_ids_batch.reshape(BATCH)
    out = _sc_onehot(ids)
    return out.reshape(BATCH, VOCAB)
